# Initial kernel scaffold; baseline (speedup 1.0000x reference)
import jax
import jax.numpy as jnp
from jax.experimental import pallas as pl


def kernel(x, edge_weights):
    raise NotImplementedError("write your pallas kernel here")


# speedup vs baseline: 5.0899x; 5.0899x over previous
import jax
import jax.numpy as jnp
from jax.experimental import pallas as pl
from jax.experimental.pallas import tpu as pltpu

_B = 4096
_D = 16
_K1 = 21
_BLOCK = 128


def _adj_kernel(xr_ref, xa_ref, w_ref, out_ref):
    xa = xa_ref[...]
    na = jnp.sqrt(jnp.sum(xa * xa, axis=1, keepdims=True))
    xan = xa / jnp.maximum(na, 1e-12)
    xr = xr_ref[...]
    nr = jnp.sqrt(jnp.sum(xr * xr, axis=1, keepdims=True))
    xrn = xr / jnp.maximum(nr, 1e-12)


    sims = jax.lax.dot_general(
        xrn, xan, (((1,), (1,)), ((), ())),
        preferred_element_type=jnp.float32)

    w = w_ref[0, :]
    denom = jnp.sum(w) + 1e-8
    wn = w / denom
    widx = jax.lax.iota(jnp.int32, _K1)

    cols = jax.lax.broadcasted_iota(jnp.int32, (_BLOCK, _B), 1)

    def body(r, carry):
        vals, out = carry
        m = jnp.max(vals, axis=1, keepdims=True)
        midx = jnp.where(vals == m, cols, _B)
        amin = jnp.min(midx, axis=1, keepdims=True)
        sel = midx == amin
        wr = jnp.sum(jnp.where(widx == r, wn, 0.0))
        out = jnp.where(sel, wr, out)
        vals = jnp.where(sel, -2.0, vals)
        return vals, out

    out0 = jnp.zeros((_BLOCK, _B), dtype=jnp.float32)
    _, out = jax.lax.fori_loop(0, _K1, body, (sims, out0))
    out_ref[...] = out


def kernel(x, edge_weights):
    w2d = edge_weights.reshape(1, _K1)
    return pl.pallas_call(
        _adj_kernel,
        grid=(_B // _BLOCK,),
        in_specs=[
            pl.BlockSpec((_BLOCK, _D), lambda i: (i, 0)),
            pl.BlockSpec((_B, _D), lambda i: (0, 0)),
            pl.BlockSpec((1, _K1), lambda i: (0, 0)),
        ],
        out_specs=pl.BlockSpec((_BLOCK, _B), lambda i: (i, 0)),
        out_shape=jax.ShapeDtypeStruct((_B, _B), jnp.float32),
    )(x, x, w2d)
